# baseline (speedup 1.0000x reference)
import functools

import jax
import jax.numpy as jnp
from jax import lax
from jax.experimental import pallas as pl
from jax.experimental.pallas import tpu as pltpu
from jax.experimental.pallas import tpu_sc as plsc

_N = 10000
_E = 160000
_D = 256
_C = 40
_HALF = 128
_NC = 2
_NS = 16
_NPAD = 10240
_K = 125
_NCH = 80
_G = 40
_NG = _NCH // _G
_RPT = _NPAD // _NS
_RB = 1000


def _segsum_sc(h_flat, src_idx, dst_idx, zeros):
    mesh = plsc.VectorSubcoreMesh(core_axis_name="c", subcore_axis_name="s")

    @functools.partial(
        pl.kernel,
        mesh=mesh,
        out_type=jax.ShapeDtypeStruct((_NC * _NPAD, _HALF), jnp.float32),
        scratch_types=[
            pltpu.VMEM((_G, _K), jnp.int32),
            pltpu.VMEM((_G, _K), jnp.int32),
            pltpu.VMEM((_K, _HALF), jnp.float32),
            pltpu.VMEM((_K, _HALF), jnp.float32),
            pltpu.VMEM_SHARED((_NPAD, _HALF), jnp.float32),
        ],
    )
    def seg(h_hbm, src_hbm, dst_hbm, z_hbm, out_hbm,
            srcv, dstv, buf0, buf1, acc):
        c = lax.axis_index("c")
        s = lax.axis_index("s")
        pltpu.sync_copy(z_hbm.at[pl.ds(s * _RPT, _RPT)],
                        acc.at[pl.ds(s * _RPT, _RPT)])
        plsc.subcore_barrier()

        @functools.partial(pl.run_scoped,
                           sem0=pltpu.SemaphoreType.DMA,
                           sem1=pltpu.SemaphoreType.DMA,
                           ssem0=pltpu.SemaphoreType.DMA,
                           ssem1=pltpu.SemaphoreType.DMA)
        def _(sem0, sem1, ssem0, ssem1):
            def gather(j, buf, sem):
                return pltpu.make_async_copy(h_hbm.at[srcv.at[j]], buf, sem)

            def scatter(j, buf, sem):
                return pltpu.make_async_copy(buf, acc.at[dstv.at[j]], sem)

            def group(g, carry):
                pltpu.sync_copy(src_hbm.at[c, s, pl.ds(g * _G, _G)], srcv)
                pltpu.sync_copy(dst_hbm.at[s, pl.ds(g * _G, _G)], dstv)
                gather(0, buf0, sem0).start()
                gather(1, buf1, sem1).start()

                def pair(i, carry):
                    j0 = 2 * i
                    gather(j0, buf0, sem0).wait()
                    scatter(j0, buf0, ssem0).start(add=True)
                    gather(j0 + 1, buf1, sem1).wait()
                    scatter(j0 + 1, buf1, ssem1).start(add=True)
                    scatter(j0, buf0, ssem0).wait()
                    gather(lax.min(j0 + 2, _G - 1), buf0, sem0).start()
                    scatter(j0 + 1, buf1, ssem1).wait()
                    gather(lax.min(j0 + 3, _G - 1), buf1, sem1).start()
                    return carry

                lax.fori_loop(0, _G // 2, pair, 0)
                gather(_G - 1, buf0, sem0).wait()
                gather(_G - 1, buf1, sem1).wait()
                return carry

            lax.fori_loop(0, _NG, group, 0)

        plsc.subcore_barrier()
        pltpu.sync_copy(acc.at[pl.ds(s * _RPT, _RPT)],
                        out_hbm.at[pl.ds(c * _NPAD + s * _RPT, _RPT)])

    return seg(h_flat, src_idx, dst_idx, zeros)


def _tc_layer(h_split, agg_split, w_split, b_row):
    def body(h_ref, a_ref, w_ref, b_ref, o_ref):
        x0 = h_ref[0] + a_ref[0]
        x1 = h_ref[1] + a_ref[1]
        z = jnp.dot(x0, w_ref[0], preferred_element_type=jnp.float32)
        z = z + jnp.dot(x1, w_ref[1], preferred_element_type=jnp.float32)
        z = jnp.maximum(z + b_ref[0], 0.0)
        o_ref[0] = z[:, :_HALF]
        o_ref[1] = z[:, _HALF:]

    return pl.pallas_call(
        body,
        grid=(_N // _RB,),
        in_specs=[
            pl.BlockSpec((_NC, _RB, _HALF), lambda i: (0, i, 0)),
            pl.BlockSpec((_NC, _RB, _HALF), lambda i: (0, i, 0)),
            pl.BlockSpec((_NC, _HALF, _D), lambda i: (0, 0, 0)),
            pl.BlockSpec((1, _D), lambda i: (0, 0)),
        ],
        out_specs=pl.BlockSpec((_NC, _RB, _HALF), lambda i: (0, i, 0)),
        out_shape=jax.ShapeDtypeStruct((_NC, _NPAD, _HALF), jnp.float32),
    )(h_split, agg_split, w_split, b_row)


def _tc_final(h_split, agg_split, w_split, b_row, wfc, bfc_row):
    def body(h_ref, a_ref, w_ref, b_ref, wfc_ref, bfc_ref, o_ref):
        x0 = h_ref[0] + a_ref[0]
        x1 = h_ref[1] + a_ref[1]
        z = jnp.dot(x0, w_ref[0], preferred_element_type=jnp.float32)
        z = z + jnp.dot(x1, w_ref[1], preferred_element_type=jnp.float32)
        z = jnp.maximum(z + b_ref[0], 0.0)
        o_ref[...] = (jnp.dot(z, wfc_ref[...], preferred_element_type=jnp.float32)
                      + bfc_ref[0])

    return pl.pallas_call(
        body,
        grid=(_N // _RB,),
        in_specs=[
            pl.BlockSpec((_NC, _RB, _HALF), lambda i: (0, i, 0)),
            pl.BlockSpec((_NC, _RB, _HALF), lambda i: (0, i, 0)),
            pl.BlockSpec((_NC, _HALF, _D), lambda i: (0, 0, 0)),
            pl.BlockSpec((1, _D), lambda i: (0, 0)),
            pl.BlockSpec((_D, _C), lambda i: (0, 0)),
            pl.BlockSpec((1, _C), lambda i: (0, 0)),
        ],
        out_specs=pl.BlockSpec((_RB, _C), lambda i: (i, 0)),
        out_shape=jax.ShapeDtypeStruct((_N, _C), jnp.float32),
    )(h_split, agg_split, w_split, b_row, wfc, bfc_row)


def _tc_layer0(x, agg_split, w_split, b_row):
    def body(h_ref, a_ref, w_ref, b_ref, o_ref):
        hx = h_ref[...]
        x0 = hx[:, :_HALF] + a_ref[0]
        x1 = hx[:, _HALF:] + a_ref[1]
        z = jnp.dot(x0, w_ref[0], preferred_element_type=jnp.float32)
        z = z + jnp.dot(x1, w_ref[1], preferred_element_type=jnp.float32)
        z = jnp.maximum(z + b_ref[0], 0.0)
        o_ref[0] = z[:, :_HALF]
        o_ref[1] = z[:, _HALF:]

    return pl.pallas_call(
        body,
        grid=(_N // _RB,),
        in_specs=[
            pl.BlockSpec((_RB, _D), lambda i: (i, 0)),
            pl.BlockSpec((_NC, _RB, _HALF), lambda i: (0, i, 0)),
            pl.BlockSpec((_NC, _HALF, _D), lambda i: (0, 0, 0)),
            pl.BlockSpec((1, _D), lambda i: (0, 0)),
        ],
        out_specs=pl.BlockSpec((_NC, _RB, _HALF), lambda i: (0, i, 0)),
        out_shape=jax.ShapeDtypeStruct((_NC, _NPAD, _HALF), jnp.float32),
    )(x, agg_split, w_split, b_row)


def kernel(x, edge_index, W0, b0, W1, b1, W2, b2, W3, b3, Wfc, bfc):
    src = edge_index[0]
    dst = edge_index[1]
    srcg = jnp.reshape(jnp.stack([src, src + _NPAD]), (_NC, _NS, _NCH, _K))
    dstg = jnp.reshape(dst, (_NS, _NCH, _K))
    zeros = jnp.zeros((_NPAD, _HALF), jnp.float32)

    srcg0 = jnp.reshape(jnp.stack([2 * src, 2 * src + 1]),
                        (_NC, _NS, _NCH, _K))
    agg = _segsum_sc(jnp.reshape(x, (_NC * _N, _HALF)), srcg0, dstg, zeros)
    h = jnp.reshape(
        _tc_layer0(x,
                   jnp.reshape(agg, (_NC, _NPAD, _HALF)),
                   jnp.reshape(W0, (_NC, _HALF, _D)),
                   jnp.reshape(b0, (1, _D))),
        (_NC * _NPAD, _HALF))

    for W, b in ((W1, b1), (W2, b2)):
        agg = _segsum_sc(h, srcg, dstg, zeros)
        h = jnp.reshape(
            _tc_layer(jnp.reshape(h, (_NC, _NPAD, _HALF)),
                      jnp.reshape(agg, (_NC, _NPAD, _HALF)),
                      jnp.reshape(W, (_NC, _HALF, _D)),
                      jnp.reshape(b, (1, _D))),
            (_NC * _NPAD, _HALF))

    agg = _segsum_sc(h, srcg, dstg, zeros)
    return _tc_final(jnp.reshape(h, (_NC, _NPAD, _HALF)),
                     jnp.reshape(agg, (_NC, _NPAD, _HALF)),
                     jnp.reshape(W3, (_NC, _HALF, _D)),
                     jnp.reshape(b3, (1, _D)),
                     Wfc,
                     jnp.reshape(bfc, (1, _C)))


# speedup vs baseline: 1.2937x; 1.2937x over previous
import functools

import jax
import jax.numpy as jnp
from jax import lax
from jax.experimental import pallas as pl
from jax.experimental.pallas import tpu as pltpu
from jax.experimental.pallas import tpu_sc as plsc

_N = 10000
_E = 160000
_D = 256
_C = 40
_HALF = 128
_NC = 2
_NS = 16
_NPAD = 10240
_K = 125
_NCH = 80
_G = 40
_NG = _NCH // _G
_RPT = _NPAD // _NS
_RB = 1000


def _segsum_sc(h_flat, src_idx, dst_idx, zeros):
    mesh = plsc.VectorSubcoreMesh(core_axis_name="c", subcore_axis_name="s")

    @functools.partial(
        pl.kernel,
        mesh=mesh,
        out_type=jax.ShapeDtypeStruct((_NC * _NPAD, _HALF), jnp.float32),
        scratch_types=[
            pltpu.VMEM((_G, _K), jnp.int32),
            pltpu.VMEM((_G, _K), jnp.int32),
            pltpu.VMEM((_K, _HALF), jnp.float32),
            pltpu.VMEM((_K, _HALF), jnp.float32),
            pltpu.VMEM_SHARED((_NPAD, _HALF), jnp.float32),
        ],
    )
    def seg(h_hbm, src_hbm, dst_hbm, z_hbm, out_hbm,
            srcv, dstv, buf0, buf1, acc):
        c = lax.axis_index("c")
        s = lax.axis_index("s")
        pltpu.sync_copy(z_hbm.at[pl.ds(s * _RPT, _RPT)],
                        acc.at[pl.ds(s * _RPT, _RPT)])
        plsc.subcore_barrier()

        @functools.partial(pl.run_scoped,
                           sem0=pltpu.SemaphoreType.DMA,
                           sem1=pltpu.SemaphoreType.DMA)
        def _(sem0, sem1):
            def gather(j, buf, sem):
                return pltpu.make_async_copy(h_hbm.at[srcv.at[j]], buf, sem)

            def group(g, carry):
                pltpu.sync_copy(src_hbm.at[c, s, pl.ds(g * _G, _G)], srcv)
                pltpu.sync_copy(dst_hbm.at[s, pl.ds(g * _G, _G)], dstv)
                gather(0, buf0, sem0).start()

                def pair(i, carry):
                    j0 = 2 * i
                    gather(j0, buf0, sem0).wait()
                    gather(j0 + 1, buf1, sem1).start()
                    pltpu.sync_copy(buf0, acc.at[dstv.at[j0]], add=True)
                    gather(lax.min(j0 + 2, _G - 1), buf0, sem0).start()
                    gather(j0 + 1, buf1, sem1).wait()
                    pltpu.sync_copy(buf1, acc.at[dstv.at[j0 + 1]], add=True)
                    return carry

                lax.fori_loop(0, _G // 2, pair, 0)
                gather(_G - 1, buf0, sem0).wait()
                return carry

            lax.fori_loop(0, _NG, group, 0)

        plsc.subcore_barrier()
        pltpu.sync_copy(acc.at[pl.ds(s * _RPT, _RPT)],
                        out_hbm.at[pl.ds(c * _NPAD + s * _RPT, _RPT)])

    return seg(h_flat, src_idx, dst_idx, zeros)


def _tc_layer(h_split, agg_split, w_split, b_row):
    def body(h_ref, a_ref, w_ref, b_ref, o_ref):
        x0 = h_ref[0] + a_ref[0]
        x1 = h_ref[1] + a_ref[1]
        z = jnp.dot(x0, w_ref[0], preferred_element_type=jnp.float32)
        z = z + jnp.dot(x1, w_ref[1], preferred_element_type=jnp.float32)
        z = jnp.maximum(z + b_ref[0], 0.0)
        o_ref[0] = z[:, :_HALF]
        o_ref[1] = z[:, _HALF:]

    return pl.pallas_call(
        body,
        grid=(_N // _RB,),
        in_specs=[
            pl.BlockSpec((_NC, _RB, _HALF), lambda i: (0, i, 0)),
            pl.BlockSpec((_NC, _RB, _HALF), lambda i: (0, i, 0)),
            pl.BlockSpec((_NC, _HALF, _D), lambda i: (0, 0, 0)),
            pl.BlockSpec((1, _D), lambda i: (0, 0)),
        ],
        out_specs=pl.BlockSpec((_NC, _RB, _HALF), lambda i: (0, i, 0)),
        out_shape=jax.ShapeDtypeStruct((_NC, _NPAD, _HALF), jnp.float32),
    )(h_split, agg_split, w_split, b_row)


def _tc_final(h_split, agg_split, w_split, b_row, wfc, bfc_row):
    def body(h_ref, a_ref, w_ref, b_ref, wfc_ref, bfc_ref, o_ref):
        x0 = h_ref[0] + a_ref[0]
        x1 = h_ref[1] + a_ref[1]
        z = jnp.dot(x0, w_ref[0], preferred_element_type=jnp.float32)
        z = z + jnp.dot(x1, w_ref[1], preferred_element_type=jnp.float32)
        z = jnp.maximum(z + b_ref[0], 0.0)
        o_ref[...] = (jnp.dot(z, wfc_ref[...], preferred_element_type=jnp.float32)
                      + bfc_ref[0])

    return pl.pallas_call(
        body,
        grid=(_N // _RB,),
        in_specs=[
            pl.BlockSpec((_NC, _RB, _HALF), lambda i: (0, i, 0)),
            pl.BlockSpec((_NC, _RB, _HALF), lambda i: (0, i, 0)),
            pl.BlockSpec((_NC, _HALF, _D), lambda i: (0, 0, 0)),
            pl.BlockSpec((1, _D), lambda i: (0, 0)),
            pl.BlockSpec((_D, _C), lambda i: (0, 0)),
            pl.BlockSpec((1, _C), lambda i: (0, 0)),
        ],
        out_specs=pl.BlockSpec((_RB, _C), lambda i: (i, 0)),
        out_shape=jax.ShapeDtypeStruct((_N, _C), jnp.float32),
    )(h_split, agg_split, w_split, b_row, wfc, bfc_row)


def _tc_layer0(x, agg_split, w_split, b_row):
    def body(h_ref, a_ref, w_ref, b_ref, o_ref):
        hx = h_ref[...]
        x0 = hx[:, :_HALF] + a_ref[0]
        x1 = hx[:, _HALF:] + a_ref[1]
        z = jnp.dot(x0, w_ref[0], preferred_element_type=jnp.float32)
        z = z + jnp.dot(x1, w_ref[1], preferred_element_type=jnp.float32)
        z = jnp.maximum(z + b_ref[0], 0.0)
        o_ref[0] = z[:, :_HALF]
        o_ref[1] = z[:, _HALF:]

    return pl.pallas_call(
        body,
        grid=(_N // _RB,),
        in_specs=[
            pl.BlockSpec((_RB, _D), lambda i: (i, 0)),
            pl.BlockSpec((_NC, _RB, _HALF), lambda i: (0, i, 0)),
            pl.BlockSpec((_NC, _HALF, _D), lambda i: (0, 0, 0)),
            pl.BlockSpec((1, _D), lambda i: (0, 0)),
        ],
        out_specs=pl.BlockSpec((_NC, _RB, _HALF), lambda i: (0, i, 0)),
        out_shape=jax.ShapeDtypeStruct((_NC, _NPAD, _HALF), jnp.float32),
    )(x, agg_split, w_split, b_row)


def kernel(x, edge_index, W0, b0, W1, b1, W2, b2, W3, b3, Wfc, bfc):
    src = edge_index[0]
    dst = edge_index[1]
    srcg = jnp.reshape(jnp.stack([src, src + _NPAD]), (_NC, _NS, _NCH, _K))
    dstg = jnp.reshape(dst, (_NS, _NCH, _K))
    zeros = jnp.zeros((_NPAD, _HALF), jnp.float32)

    srcg0 = jnp.reshape(jnp.stack([2 * src, 2 * src + 1]),
                        (_NC, _NS, _NCH, _K))
    agg = _segsum_sc(jnp.reshape(x, (_NC * _N, _HALF)), srcg0, dstg, zeros)
    h = jnp.reshape(
        _tc_layer0(x,
                   jnp.reshape(agg, (_NC, _NPAD, _HALF)),
                   jnp.reshape(W0, (_NC, _HALF, _D)),
                   jnp.reshape(b0, (1, _D))),
        (_NC * _NPAD, _HALF))

    for W, b in ((W1, b1), (W2, b2)):
        agg = _segsum_sc(h, srcg, dstg, zeros)
        h = jnp.reshape(
            _tc_layer(jnp.reshape(h, (_NC, _NPAD, _HALF)),
                      jnp.reshape(agg, (_NC, _NPAD, _HALF)),
                      jnp.reshape(W, (_NC, _HALF, _D)),
                      jnp.reshape(b, (1, _D))),
            (_NC * _NPAD, _HALF))

    agg = _segsum_sc(h, srcg, dstg, zeros)
    return _tc_final(jnp.reshape(h, (_NC, _NPAD, _HALF)),
                     jnp.reshape(agg, (_NC, _NPAD, _HALF)),
                     jnp.reshape(W3, (_NC, _HALF, _D)),
                     jnp.reshape(b3, (1, _D)),
                     Wfc,
                     jnp.reshape(bfc, (1, _C)))


# speedup vs baseline: 1.3047x; 1.0085x over previous
import functools

import jax
import jax.numpy as jnp
from jax import lax
from jax.experimental import pallas as pl
from jax.experimental.pallas import tpu as pltpu
from jax.experimental.pallas import tpu_sc as plsc

_N = 10000
_E = 160000
_D = 256
_C = 40
_HALF = 128
_NC = 2
_NS = 16
_NPAD = 10240
_K = 125
_NCH = 80
_G = 40
_NG = _NCH // _G
_RPT = _NPAD // _NS
_RB = 1000


def _segsum_sc(h_flat, src_idx, dst_idx, zeros):
    mesh = plsc.VectorSubcoreMesh(core_axis_name="c", subcore_axis_name="s")

    @functools.partial(
        pl.kernel,
        mesh=mesh,
        out_type=jax.ShapeDtypeStruct((_NC * _NPAD, _HALF), jnp.float32),
        scratch_types=[
            pltpu.VMEM((_G, _K), jnp.int32),
            pltpu.VMEM((_G, _K), jnp.int32),
            pltpu.VMEM((_K, _HALF), jnp.float32),
            pltpu.VMEM((_K, _HALF), jnp.float32),
            pltpu.VMEM_SHARED((_NPAD, _HALF), jnp.float32),
        ],
    )
    def seg(h_hbm, src_hbm, dst_hbm, z_hbm, out_hbm,
            srcv, dstv, buf0, buf1, acc):
        c = lax.axis_index("c")
        s = lax.axis_index("s")

        @functools.partial(pl.run_scoped, zsem=pltpu.SemaphoreType.DMA)
        def _(zsem):
            zero = pltpu.make_async_copy(z_hbm.at[pl.ds(s * _RPT, _RPT)],
                                         acc.at[pl.ds(s * _RPT, _RPT)], zsem)
            zero.start()
            pltpu.sync_copy(src_hbm.at[c, s, pl.ds(0, _G)], srcv)
            pltpu.sync_copy(dst_hbm.at[s, pl.ds(0, _G)], dstv)
            zero.wait()

        plsc.subcore_barrier()

        @functools.partial(pl.run_scoped,
                           sem0=pltpu.SemaphoreType.DMA,
                           sem1=pltpu.SemaphoreType.DMA)
        def _(sem0, sem1):
            def gather(j, buf, sem):
                return pltpu.make_async_copy(h_hbm.at[srcv.at[j]], buf, sem)

            def group(g, carry):
                @pl.when(g > 0)
                def _():
                    pltpu.sync_copy(src_hbm.at[c, s, pl.ds(g * _G, _G)],
                                    srcv)
                    pltpu.sync_copy(dst_hbm.at[s, pl.ds(g * _G, _G)], dstv)
                gather(0, buf0, sem0).start()

                def pair(i, carry):
                    j0 = 2 * i
                    gather(j0, buf0, sem0).wait()
                    gather(j0 + 1, buf1, sem1).start()
                    pltpu.sync_copy(buf0, acc.at[dstv.at[j0]], add=True)
                    gather(lax.min(j0 + 2, _G - 1), buf0, sem0).start()
                    gather(j0 + 1, buf1, sem1).wait()
                    pltpu.sync_copy(buf1, acc.at[dstv.at[j0 + 1]], add=True)
                    return carry

                lax.fori_loop(0, _G // 2, pair, 0)
                gather(_G - 1, buf0, sem0).wait()
                return carry

            lax.fori_loop(0, _NG, group, 0)

        plsc.subcore_barrier()
        pltpu.sync_copy(acc.at[pl.ds(s * _RPT, _RPT)],
                        out_hbm.at[pl.ds(c * _NPAD + s * _RPT, _RPT)])

    return seg(h_flat, src_idx, dst_idx, zeros)


def _tc_layer(h_split, agg_split, w_split, b_row):
    def body(h_ref, a_ref, w_ref, b_ref, o_ref):
        x0 = h_ref[0] + a_ref[0]
        x1 = h_ref[1] + a_ref[1]
        z = jnp.dot(x0, w_ref[0], preferred_element_type=jnp.float32)
        z = z + jnp.dot(x1, w_ref[1], preferred_element_type=jnp.float32)
        z = jnp.maximum(z + b_ref[0], 0.0)
        o_ref[0] = z[:, :_HALF]
        o_ref[1] = z[:, _HALF:]

    return pl.pallas_call(
        body,
        grid=(_N // _RB,),
        in_specs=[
            pl.BlockSpec((_NC, _RB, _HALF), lambda i: (0, i, 0)),
            pl.BlockSpec((_NC, _RB, _HALF), lambda i: (0, i, 0)),
            pl.BlockSpec((_NC, _HALF, _D), lambda i: (0, 0, 0)),
            pl.BlockSpec((1, _D), lambda i: (0, 0)),
        ],
        out_specs=pl.BlockSpec((_NC, _RB, _HALF), lambda i: (0, i, 0)),
        out_shape=jax.ShapeDtypeStruct((_NC, _NPAD, _HALF), jnp.float32),
    )(h_split, agg_split, w_split, b_row)


def _tc_final(h_split, agg_split, w_split, b_row, wfc, bfc_row):
    def body(h_ref, a_ref, w_ref, b_ref, wfc_ref, bfc_ref, o_ref):
        x0 = h_ref[0] + a_ref[0]
        x1 = h_ref[1] + a_ref[1]
        z = jnp.dot(x0, w_ref[0], preferred_element_type=jnp.float32)
        z = z + jnp.dot(x1, w_ref[1], preferred_element_type=jnp.float32)
        z = jnp.maximum(z + b_ref[0], 0.0)
        o_ref[...] = (jnp.dot(z, wfc_ref[...], preferred_element_type=jnp.float32)
                      + bfc_ref[0])

    return pl.pallas_call(
        body,
        grid=(_N // _RB,),
        in_specs=[
            pl.BlockSpec((_NC, _RB, _HALF), lambda i: (0, i, 0)),
            pl.BlockSpec((_NC, _RB, _HALF), lambda i: (0, i, 0)),
            pl.BlockSpec((_NC, _HALF, _D), lambda i: (0, 0, 0)),
            pl.BlockSpec((1, _D), lambda i: (0, 0)),
            pl.BlockSpec((_D, _C), lambda i: (0, 0)),
            pl.BlockSpec((1, _C), lambda i: (0, 0)),
        ],
        out_specs=pl.BlockSpec((_RB, _C), lambda i: (i, 0)),
        out_shape=jax.ShapeDtypeStruct((_N, _C), jnp.float32),
    )(h_split, agg_split, w_split, b_row, wfc, bfc_row)


def _tc_layer0(x, agg_split, w_split, b_row):
    def body(h_ref, a_ref, w_ref, b_ref, o_ref):
        hx = h_ref[...]
        x0 = hx[:, :_HALF] + a_ref[0]
        x1 = hx[:, _HALF:] + a_ref[1]
        z = jnp.dot(x0, w_ref[0], preferred_element_type=jnp.float32)
        z = z + jnp.dot(x1, w_ref[1], preferred_element_type=jnp.float32)
        z = jnp.maximum(z + b_ref[0], 0.0)
        o_ref[0] = z[:, :_HALF]
        o_ref[1] = z[:, _HALF:]

    return pl.pallas_call(
        body,
        grid=(_N // _RB,),
        in_specs=[
            pl.BlockSpec((_RB, _D), lambda i: (i, 0)),
            pl.BlockSpec((_NC, _RB, _HALF), lambda i: (0, i, 0)),
            pl.BlockSpec((_NC, _HALF, _D), lambda i: (0, 0, 0)),
            pl.BlockSpec((1, _D), lambda i: (0, 0)),
        ],
        out_specs=pl.BlockSpec((_NC, _RB, _HALF), lambda i: (0, i, 0)),
        out_shape=jax.ShapeDtypeStruct((_NC, _NPAD, _HALF), jnp.float32),
    )(x, agg_split, w_split, b_row)


def kernel(x, edge_index, W0, b0, W1, b1, W2, b2, W3, b3, Wfc, bfc):
    src = edge_index[0]
    dst = edge_index[1]
    srcg = jnp.reshape(jnp.stack([src, src + _NPAD]), (_NC, _NS, _NCH, _K))
    dstg = jnp.reshape(dst, (_NS, _NCH, _K))
    zeros = jnp.zeros((_NPAD, _HALF), jnp.float32)

    srcg0 = jnp.reshape(jnp.stack([2 * src, 2 * src + 1]),
                        (_NC, _NS, _NCH, _K))
    agg = _segsum_sc(jnp.reshape(x, (_NC * _N, _HALF)), srcg0, dstg, zeros)
    h = jnp.reshape(
        _tc_layer0(x,
                   jnp.reshape(agg, (_NC, _NPAD, _HALF)),
                   jnp.reshape(W0, (_NC, _HALF, _D)),
                   jnp.reshape(b0, (1, _D))),
        (_NC * _NPAD, _HALF))

    for W, b in ((W1, b1), (W2, b2)):
        agg = _segsum_sc(h, srcg, dstg, zeros)
        h = jnp.reshape(
            _tc_layer(jnp.reshape(h, (_NC, _NPAD, _HALF)),
                      jnp.reshape(agg, (_NC, _NPAD, _HALF)),
                      jnp.reshape(W, (_NC, _HALF, _D)),
                      jnp.reshape(b, (1, _D))),
            (_NC * _NPAD, _HALF))

    agg = _segsum_sc(h, srcg, dstg, zeros)
    return _tc_final(jnp.reshape(h, (_NC, _NPAD, _HALF)),
                     jnp.reshape(agg, (_NC, _NPAD, _HALF)),
                     jnp.reshape(W3, (_NC, _HALF, _D)),
                     jnp.reshape(b3, (1, _D)),
                     Wfc,
                     jnp.reshape(bfc, (1, _C)))
